# baseline (speedup 1.0000x reference)
import functools

import jax
import jax.numpy as jnp
from jax import lax
from jax.experimental import pallas as pl
from jax.experimental.pallas import tpu as pltpu
from jax.experimental.pallas import tpu_sc as plsc

_RANK = 64
_D = 2 * _RANK
_B = 1024
_NW = 32
_BPW = _B // _NW
_L = 16


def _combine_body(entity_hbm, relation_hbm, qlhs_hbm, qrel_hbm, c_hbm,
                  idx_l, idx_r, lhs_v, rel_v, sem):
    wid = lax.axis_index("s") * 2 + lax.axis_index("c")
    base = wid * _BPW
    pltpu.sync_copy(qlhs_hbm.at[pl.ds(base, _BPW)], idx_l)
    pltpu.sync_copy(qrel_hbm.at[pl.ds(base, _BPW)], idx_r)
    cp1 = pltpu.async_copy(entity_hbm.at[idx_l], lhs_v, sem)
    cp2 = pltpu.async_copy(relation_hbm.at[idx_r], rel_v, sem)
    cp1.wait()
    cp2.wait()

    def row(i, _):
        for j in range(_RANK // _L):
            re_sl = pl.ds(j * _L, _L)
            im_sl = pl.ds(_RANK + j * _L, _L)
            a_re = lhs_v[i, re_sl]
            a_im = lhs_v[i, im_sl]
            r_re = rel_v[i, re_sl]
            r_im = rel_v[i, im_sl]
            lhs_v[i, re_sl] = a_re * r_re - a_im * r_im
            lhs_v[i, im_sl] = a_re * r_im + a_im * r_re
        return 0

    lax.fori_loop(0, _BPW, row, 0)
    pltpu.sync_copy(lhs_v, c_hbm.at[pl.ds(base, _BPW)])


@jax.jit
def _combine(qlhs, qrel, entity, relation):
    mesh = plsc.VectorSubcoreMesh(core_axis_name="c", subcore_axis_name="s")
    return pl.kernel(
        _combine_body,
        out_type=jax.ShapeDtypeStruct((_B, _D), jnp.float32),
        mesh=mesh,
        scratch_types=[
            pltpu.VMEM((_BPW,), jnp.int32),
            pltpu.VMEM((_BPW,), jnp.int32),
            pltpu.VMEM((_BPW, _D), jnp.float32),
            pltpu.VMEM((_BPW, _D), jnp.float32),
            pltpu.SemaphoreType.DMA,
        ],
    )(entity, relation, qlhs, qrel)


_NBUF = 4


def _mm_body(c_ref, e_ref, o_hbm, acc, sems, *, v_blk, n_e):
    grid = n_e // v_blk
    last = grid - 1
    i = pl.program_id(0)
    slot = lax.rem(i, _NBUF)

    @pl.when(i >= _NBUF)
    def _wait_prev():
        pltpu.make_async_copy(
            acc.at[slot],
            o_hbm.at[pl.ds((i - _NBUF) * v_blk, v_blk), :],
            sems.at[slot],
        ).wait()

    acc[slot] = lax.dot_general(
        e_ref[...], c_ref[...],
        dimension_numbers=(((1,), (1,)), ((), ())),
        preferred_element_type=jnp.float32,
    )
    pltpu.make_async_copy(
        acc.at[slot],
        o_hbm.at[pl.ds(i * v_blk, v_blk), :],
        sems.at[slot],
    ).start()

    @pl.when(i == last)
    def _drain():
        for s in range(_NBUF):
            t_off = (last - s) % _NBUF
            pltpu.make_async_copy(
                acc.at[s],
                o_hbm.at[pl.ds((last - t_off) * v_blk, v_blk), :],
                sems.at[s],
            ).wait()


@functools.partial(jax.jit, static_argnames=("v_blk",))
def _score_t(c, entity, v_blk):
    n_e = entity.shape[0]
    assert n_e % v_blk == 0 and n_e // v_blk > _NBUF
    grid = n_e // v_blk
    return pl.pallas_call(
        functools.partial(_mm_body, v_blk=v_blk, n_e=n_e),
        grid=(grid,),
        in_specs=[
            pl.BlockSpec((_B, _D), lambda i: (0, 0)),
            pl.BlockSpec((v_blk, _D), lambda i: (i, 0)),
        ],
        out_specs=pl.BlockSpec(memory_space=pl.ANY),
        out_shape=jax.ShapeDtypeStruct((n_e, _B), jnp.float32),
        scratch_shapes=[
            pltpu.VMEM((_NBUF, v_blk, _B), jnp.float32),
            pltpu.SemaphoreType.DMA((_NBUF,)),
        ],
        compiler_params=pltpu.CompilerParams(
            dimension_semantics=("arbitrary",),
        ),
    )(c, entity)


def kernel(queries, entity, relation):
    qlhs = queries[:, 0].astype(jnp.int32)
    qrel = queries[:, 1].astype(jnp.int32)
    c = _combine(qlhs, qrel, entity, relation)
    return _score_t(c, entity, v_blk=1000).T


# speedup vs baseline: 1.0512x; 1.0512x over previous
import functools

import jax
import jax.numpy as jnp
from jax import lax
from jax.experimental import pallas as pl
from jax.experimental.pallas import tpu as pltpu
from jax.experimental.pallas import tpu_sc as plsc

_RANK = 64
_D = 2 * _RANK
_B = 1024
_NW = 32
_BPW = _B // _NW
_L = 16


def _combine_body(entity_hbm, relation_hbm, qlhs_hbm, qrel_hbm, c_hbm,
                  idx_l, idx_r, lhs_v, rel_v, sem):
    wid = lax.axis_index("s") * 2 + lax.axis_index("c")
    base = wid * _BPW
    pltpu.sync_copy(qlhs_hbm.at[pl.ds(base, _BPW)], idx_l)
    pltpu.sync_copy(qrel_hbm.at[pl.ds(base, _BPW)], idx_r)
    cp1 = pltpu.async_copy(entity_hbm.at[idx_l], lhs_v, sem)
    cp2 = pltpu.async_copy(relation_hbm.at[idx_r], rel_v, sem)
    cp1.wait()
    cp2.wait()

    def row(i, _):
        for j in range(_RANK // _L):
            re_sl = pl.ds(j * _L, _L)
            im_sl = pl.ds(_RANK + j * _L, _L)
            a_re = lhs_v[i, re_sl]
            a_im = lhs_v[i, im_sl]
            r_re = rel_v[i, re_sl]
            r_im = rel_v[i, im_sl]
            lhs_v[i, re_sl] = a_re * r_re - a_im * r_im
            lhs_v[i, im_sl] = a_re * r_im + a_im * r_re
        return 0

    lax.fori_loop(0, _BPW, row, 0)
    pltpu.sync_copy(lhs_v, c_hbm.at[pl.ds(base, _BPW)])


@jax.jit
def _combine(qlhs, qrel, entity, relation):
    mesh = plsc.VectorSubcoreMesh(core_axis_name="c", subcore_axis_name="s")
    return pl.kernel(
        _combine_body,
        out_type=jax.ShapeDtypeStruct((_B, _D), jnp.float32),
        mesh=mesh,
        scratch_types=[
            pltpu.VMEM((_BPW,), jnp.int32),
            pltpu.VMEM((_BPW,), jnp.int32),
            pltpu.VMEM((_BPW, _D), jnp.float32),
            pltpu.VMEM((_BPW, _D), jnp.float32),
            pltpu.SemaphoreType.DMA,
        ],
    )(entity, relation, qlhs, qrel)


_NBUF = 3


def _mm_body(c_ref, e_ref, o_hbm, acc, sems, *, v_blk, n_e):
    grid = n_e // v_blk
    last = grid - 1
    i = pl.program_id(0)
    slot = lax.rem(i, _NBUF)

    @pl.when(i >= _NBUF)
    def _wait_prev():
        pltpu.make_async_copy(
            acc.at[slot],
            o_hbm.at[pl.ds((i - _NBUF) * v_blk, v_blk), :],
            sems.at[slot],
        ).wait()

    acc[slot] = lax.dot_general(
        e_ref[...], c_ref[...],
        dimension_numbers=(((1,), (1,)), ((), ())),
        preferred_element_type=jnp.float32,
    )
    pltpu.make_async_copy(
        acc.at[slot],
        o_hbm.at[pl.ds(i * v_blk, v_blk), :],
        sems.at[slot],
    ).start()

    @pl.when(i == last)
    def _drain():
        for s in range(_NBUF):
            t_off = (last - s) % _NBUF
            pltpu.make_async_copy(
                acc.at[s],
                o_hbm.at[pl.ds((last - t_off) * v_blk, v_blk), :],
                sems.at[s],
            ).wait()


@functools.partial(jax.jit, static_argnames=("v_blk",))
def _score_t(c, entity, v_blk):
    n_e = entity.shape[0]
    assert n_e % v_blk == 0 and n_e // v_blk > _NBUF
    grid = n_e // v_blk
    return pl.pallas_call(
        functools.partial(_mm_body, v_blk=v_blk, n_e=n_e),
        grid=(grid,),
        in_specs=[
            pl.BlockSpec((_B, _D), lambda i: (0, 0)),
            pl.BlockSpec((v_blk, _D), lambda i: (i, 0)),
        ],
        out_specs=pl.BlockSpec(memory_space=pl.ANY),
        out_shape=jax.ShapeDtypeStruct((n_e, _B), jnp.float32),
        scratch_shapes=[
            pltpu.VMEM((_NBUF, v_blk, _B), jnp.float32),
            pltpu.SemaphoreType.DMA((_NBUF,)),
        ],
        compiler_params=pltpu.CompilerParams(
            dimension_semantics=("arbitrary",),
        ),
    )(c, entity)


def kernel(queries, entity, relation):
    qlhs = queries[:, 0].astype(jnp.int32)
    qrel = queries[:, 1].astype(jnp.int32)
    c = _combine(qlhs, qrel, entity, relation)
    return _score_t(c, entity, v_blk=4000).T


# speedup vs baseline: 1.0550x; 1.0037x over previous
import functools

import jax
import jax.numpy as jnp
from jax import lax
from jax.experimental import pallas as pl
from jax.experimental.pallas import tpu as pltpu
from jax.experimental.pallas import tpu_sc as plsc

_RANK = 64
_D = 2 * _RANK
_B = 1024
_NW = 32
_BPW = _B // _NW
_L = 16


def _combine_body(entity_hbm, relation_hbm, qlhs_hbm, qrel_hbm, c_hbm,
                  idx_l, idx_r, lhs_v, rel_v, sem):
    wid = lax.axis_index("s") * 2 + lax.axis_index("c")
    base = wid * _BPW
    pltpu.sync_copy(qlhs_hbm.at[pl.ds(base, _BPW)], idx_l)
    pltpu.sync_copy(qrel_hbm.at[pl.ds(base, _BPW)], idx_r)
    cp1 = pltpu.async_copy(entity_hbm.at[idx_l], lhs_v, sem)
    cp2 = pltpu.async_copy(relation_hbm.at[idx_r], rel_v, sem)
    cp1.wait()
    cp2.wait()

    def row(i, _):
        for j in range(_RANK // _L):
            re_sl = pl.ds(j * _L, _L)
            im_sl = pl.ds(_RANK + j * _L, _L)
            a_re = lhs_v[i, re_sl]
            a_im = lhs_v[i, im_sl]
            r_re = rel_v[i, re_sl]
            r_im = rel_v[i, im_sl]
            lhs_v[i, re_sl] = a_re * r_re - a_im * r_im
            lhs_v[i, im_sl] = a_re * r_im + a_im * r_re
        return 0

    lax.fori_loop(0, _BPW, row, 0)
    pltpu.sync_copy(lhs_v, c_hbm.at[pl.ds(base, _BPW)])


@jax.jit
def _combine(qlhs, qrel, entity, relation):
    mesh = plsc.VectorSubcoreMesh(core_axis_name="c", subcore_axis_name="s")
    return pl.kernel(
        _combine_body,
        out_type=jax.ShapeDtypeStruct((_B, _D), jnp.float32),
        mesh=mesh,
        scratch_types=[
            pltpu.VMEM((_BPW,), jnp.int32),
            pltpu.VMEM((_BPW,), jnp.int32),
            pltpu.VMEM((_BPW, _D), jnp.float32),
            pltpu.VMEM((_BPW, _D), jnp.float32),
            pltpu.SemaphoreType.DMA,
        ],
    )(entity, relation, qlhs, qrel)


_NBUF = 2


def _mm_body(c_ref, e_ref, o_hbm, acc, sems, *, v_blk, n_e):
    grid = n_e // v_blk
    last = grid - 1
    i = pl.program_id(0)
    slot = lax.rem(i, _NBUF)

    @pl.when(i >= _NBUF)
    def _wait_prev():
        pltpu.make_async_copy(
            acc.at[slot],
            o_hbm.at[pl.ds((i - _NBUF) * v_blk, v_blk), :],
            sems.at[slot],
        ).wait()

    acc[slot] = lax.dot_general(
        e_ref[...], c_ref[...],
        dimension_numbers=(((1,), (1,)), ((), ())),
        preferred_element_type=jnp.float32,
    )
    pltpu.make_async_copy(
        acc.at[slot],
        o_hbm.at[pl.ds(i * v_blk, v_blk), :],
        sems.at[slot],
    ).start()

    @pl.when(i == last)
    def _drain():
        for s in range(_NBUF):
            t_off = (last - s) % _NBUF
            pltpu.make_async_copy(
                acc.at[s],
                o_hbm.at[pl.ds((last - t_off) * v_blk, v_blk), :],
                sems.at[s],
            ).wait()


@functools.partial(jax.jit, static_argnames=("v_blk",))
def _score_t(c, entity, v_blk):
    n_e = entity.shape[0]
    assert n_e % v_blk == 0 and n_e // v_blk > _NBUF
    grid = n_e // v_blk
    return pl.pallas_call(
        functools.partial(_mm_body, v_blk=v_blk, n_e=n_e),
        grid=(grid,),
        in_specs=[
            pl.BlockSpec((_B, _D), lambda i: (0, 0)),
            pl.BlockSpec((v_blk, _D), lambda i: (i, 0)),
        ],
        out_specs=pl.BlockSpec(memory_space=pl.ANY),
        out_shape=jax.ShapeDtypeStruct((n_e, _B), jnp.float32),
        scratch_shapes=[
            pltpu.VMEM((_NBUF, v_blk, _B), jnp.float32),
            pltpu.SemaphoreType.DMA((_NBUF,)),
        ],
        compiler_params=pltpu.CompilerParams(
            dimension_semantics=("arbitrary",),
        ),
    )(c, entity)


def kernel(queries, entity, relation):
    qlhs = queries[:, 0].astype(jnp.int32)
    qrel = queries[:, 1].astype(jnp.int32)
    c = _combine(qlhs, qrel, entity, relation)
    return _score_t(c, entity, v_blk=5000).T
